# Initial kernel scaffold; baseline (speedup 1.0000x reference)
import jax
import jax.numpy as jnp
from jax.experimental import pallas as pl


def kernel(input_, keys, pre_W1, pre_b1, pre_W2, pre_b2, alpha, Wqk, Wv, Wo_s, rot, Wq_c, Wk_c, Wv_c, Wo_c, Wf1, bf1, Wf2, bf2, Wm, bm, Ws, bs):
    raise NotImplementedError("write your pallas kernel here")


# speedup vs baseline: 1.2421x; 1.2421x over previous
import functools

import jax
import jax.numpy as jnp
import numpy as np
from jax import lax
from jax.experimental import pallas as pl

B, SQ, SK = 2, 2048, 512
NMEL, D, H, DH = 80, 768, 12, 64
L, CHUNK, NB, FF, PRE_H = 2, 64, 32, 3072, 256
NC = SQ // CHUNK
RBLK = 256
RB = 256
PB = 256
F32 = jnp.float32


def _fiota(shape, dim):
    return lax.broadcasted_iota(jnp.int32, shape, dim).astype(F32)


def _ln(x):
    m = jnp.mean(x, axis=-1, keepdims=True)
    v = jnp.mean((x - m) * (x - m), axis=-1, keepdims=True)
    return (x - m) / jnp.sqrt(v + 1e-5)


HI = lax.Precision.HIGHEST


def _dot(a, b, prec=None):
    return jnp.dot(a, b, preferred_element_type=F32, precision=prec)


def _dot_tn(a, b):
    return lax.dot_general(a, b, (((0,), (0,)), ((), ())),
                           preferred_element_type=F32)


def _split3(x):
    hi = x.astype(jnp.bfloat16).astype(F32)
    mid = (x - hi).astype(jnp.bfloat16).astype(F32)
    return hi, mid, x - hi - mid


def _perm_tn(p, x):
    hi, mid, lo = _split3(x)
    return (_dot_tn(p, hi) + _dot_tn(p, mid)) + _dot_tn(p, lo)


def _perm_nn(p, x):
    hi, mid, lo = _split3(x)
    return (_dot(p, hi) + _dot(p, mid)) + _dot(p, lo)


def _dot_nt(a, b):
    return lax.dot_general(a, b, (((1,), (1,)), ((), ())),
                           preferred_element_type=F32)


def _prenet_kernel(inp_ref, w1_ref, b1_ref, w2_ref, b2_ref, pe_ref, o_ref):
    x = jnp.maximum(_dot(inp_ref[0], w1_ref[...]) + b1_ref[...], 0.0)
    x = jnp.maximum(_dot(x, w2_ref[...]) + b2_ref[...], 0.0)
    o_ref[0] = x + pe_ref[...]


def _prenet(inp, w1, b1, w2, b2, pe_scaled):
    return pl.pallas_call(
        _prenet_kernel,
        grid=(B, SQ // RB),
        in_specs=[
            pl.BlockSpec((1, RB, NMEL), lambda b, i: (b, i, 0)),
            pl.BlockSpec((NMEL, PRE_H), lambda b, i: (0, 0)),
            pl.BlockSpec((1, PRE_H), lambda b, i: (0, 0)),
            pl.BlockSpec((PRE_H, D), lambda b, i: (0, 0)),
            pl.BlockSpec((1, D), lambda b, i: (0, 0)),
            pl.BlockSpec((RB, D), lambda b, i: (i, 0)),
        ],
        out_specs=pl.BlockSpec((1, RB, D), lambda b, i: (b, i, 0)),
        out_shape=jax.ShapeDtypeStruct((B, SQ, D), F32),
    )(inp, w1, b1, w2, b2, pe_scaled)


def _proj_kernel(x_ref, wqk_ref, wv_ref, qk_ref, v_ref):
    h = _ln(x_ref[0])
    qk = _dot(h, wqk_ref[...])
    v = _dot(h, wv_ref[...])
    for hh in range(H):
        qk_ref[0, hh] = qk[:, hh * DH:(hh + 1) * DH]
        v_ref[0, hh] = v[:, hh * DH:(hh + 1) * DH]


def _proj(x, wqk, wv):
    return pl.pallas_call(
        _proj_kernel,
        grid=(B, SQ // RB),
        in_specs=[
            pl.BlockSpec((1, RB, D), lambda b, i: (b, i, 0)),
            pl.BlockSpec((D, D), lambda b, i: (0, 0)),
            pl.BlockSpec((D, D), lambda b, i: (0, 0)),
        ],
        out_specs=[
            pl.BlockSpec((1, H, RB, DH), lambda b, i: (b, 0, i, 0)),
            pl.BlockSpec((1, H, RB, DH), lambda b, i: (b, 0, i, 0)),
        ],
        out_shape=[
            jax.ShapeDtypeStruct((B, H, SQ, DH), F32),
            jax.ShapeDtypeStruct((B, H, SQ, DH), F32),
        ],
    )(x, wqk, wv)


def _lsh_kernel(qk_ref, v_ref, rk_ref, x_ref, wo_ref, o_ref):
    hh = pl.program_id(1)
    lane = _fiota((PB, SQ), 1)
    rowi = _fiota((PB, SQ), 0)
    iota8 = _fiota((PB, 8), 0)

    sqk = jnp.zeros((SQ, DH), F32)
    sv = jnp.zeros((SQ, DH), F32)
    spc = jnp.zeros((SQ, 8), F32)
    spos_r = jnp.zeros((1, SQ), F32)
    for rb in range(SQ // PB):
        r_blk = rk_ref[0, 0][rb * PB:(rb + 1) * PB, 0:1]
        p_blk = jnp.where(lane == r_blk, 1.0, 0.0)
        sqk = sqk + _perm_tn(p_blk, qk_ref[0, 0][rb * PB:(rb + 1) * PB])
        sv = sv + _perm_tn(p_blk, v_ref[0, 0][rb * PB:(rb + 1) * PB])
        spc = spc + _perm_tn(p_blk, iota8 + float(rb * PB))
        spos_r = spos_r + jnp.sum(p_blk * (rowi + float(rb * PB)),
                                  axis=0, keepdims=True)
    spos_c = spc[:, 0:1]

    nrm = jnp.sqrt(jnp.sum(sqk * sqk, axis=1, keepdims=True))
    kn = sqk / (nrm + 1e-6)

    kn_ext = jnp.concatenate([kn[SQ - CHUNK:], kn], axis=0)
    sv_ext = jnp.concatenate([sv[SQ - CHUNK:], sv], axis=0)
    sp_ext = jnp.concatenate([spos_r[:, SQ - CHUNK:], spos_r], axis=1)

    outs = []
    for c in range(NC):
        q = sqk[c * CHUNK:(c + 1) * CHUNK]
        k = kn_ext[c * CHUNK:c * CHUNK + 2 * CHUNK]
        vv = sv_ext[c * CHUNK:c * CHUNK + 2 * CHUNK]
        pq = spos_c[c * CHUNK:(c + 1) * CHUNK]
        pk = sp_ext[:, c * CHUNK:c * CHUNK + 2 * CHUNK]
        s = _dot_nt(q, k) * 0.125
        s = jnp.where(pq >= pk, s, -1e9)
        s = jnp.where(pq == pk, -1e5, s)
        m = jnp.max(s, axis=1, keepdims=True)
        e = jnp.exp(s - m)
        a = e / jnp.sum(e, axis=1, keepdims=True)
        outs.append(_dot(a, vv))
    out_s = jnp.concatenate(outs, axis=0)
    outs2 = []
    for rb in range(SQ // PB):
        r_blk = rk_ref[0, 0][rb * PB:(rb + 1) * PB, 0:1]
        p_blk = jnp.where(lane == r_blk, 1.0, 0.0)
        outs2.append(_perm_nn(p_blk, out_s))
    out = jnp.concatenate(outs2, axis=0)
    part = _dot(out, wo_ref[...])

    @pl.when(hh == 0)
    def _():
        o_ref[0] = x_ref[0] + part

    @pl.when(hh > 0)
    def _():
        o_ref[0] = o_ref[0] + part


def _lsh_attn(qk4, v4, rk, x, wo):
    return pl.pallas_call(
        _lsh_kernel,
        grid=(B, H),
        in_specs=[
            pl.BlockSpec((1, 1, SQ, DH), lambda b, h: (b, h, 0, 0)),
            pl.BlockSpec((1, 1, SQ, DH), lambda b, h: (b, h, 0, 0)),
            pl.BlockSpec((1, 1, SQ, 8), lambda b, h: (b, h, 0, 0)),
            pl.BlockSpec((1, SQ, D), lambda b, h: (b, 0, 0)),
            pl.BlockSpec((DH, D), lambda b, h: (h, 0)),
        ],
        out_specs=pl.BlockSpec((1, SQ, D), lambda b, h: (b, 0, 0)),
        out_shape=jax.ShapeDtypeStruct((B, SQ, D), F32),
    )(qk4, v4, rk, x, wo)


CQ = 512

def _cross_kernel(x_ref, h_ref, keys_ref, wq_ref, wk_ref, wv_ref, wo_ref,
                  o_ref, attn_ref):
    hh = pl.program_id(2)
    q = _dot(h_ref[0], wq_ref[0])
    k = _dot(keys_ref[0], wk_ref[0])
    v = _dot(keys_ref[0], wv_ref[0])
    s = _dot_nt(q, k) * 0.125
    m = jnp.max(s, axis=1, keepdims=True)
    e = jnp.exp(s - m)
    a = e / jnp.sum(e, axis=1, keepdims=True)
    attn_ref[0, 0] = a
    co = _dot(a, v)
    part = _dot(co, wo_ref[...])

    @pl.when(hh == 0)
    def _():
        o_ref[0] = x_ref[0] + part

    @pl.when(hh > 0)
    def _():
        o_ref[0] = o_ref[0] + part


def _cross(x, h, keys, wq, wk, wv, wo):
    return pl.pallas_call(
        _cross_kernel,
        grid=(B, SQ // CQ, H),
        in_specs=[
            pl.BlockSpec((1, CQ, D), lambda b, i, h: (b, i, 0)),
            pl.BlockSpec((1, CQ, D), lambda b, i, h: (b, i, 0)),
            pl.BlockSpec((1, SK, D), lambda b, i, h: (b, 0, 0)),
            pl.BlockSpec((1, D, DH), lambda b, i, h: (h, 0, 0)),
            pl.BlockSpec((1, D, DH), lambda b, i, h: (h, 0, 0)),
            pl.BlockSpec((1, D, DH), lambda b, i, h: (h, 0, 0)),
            pl.BlockSpec((DH, D), lambda b, i, h: (h, 0)),
        ],
        out_specs=[
            pl.BlockSpec((1, CQ, D), lambda b, i, h: (b, i, 0)),
            pl.BlockSpec((1, 1, CQ, SK), lambda b, i, h: (b, h, i, 0)),
        ],
        out_shape=[
            jax.ShapeDtypeStruct((B, SQ, D), F32),
            jax.ShapeDtypeStruct((B, H, SQ, SK), F32),
        ],
    )(x, h, keys,
      wq.reshape(D, H, DH).transpose(1, 0, 2),
      wk.reshape(D, H, DH).transpose(1, 0, 2),
      wv.reshape(D, H, DH).transpose(1, 0, 2),
      wo)


def _ffn_kernel(x_ref, h_ref, w1_ref, b1_ref, w2_ref, b2_ref, o_ref):
    x = x_ref[...]
    y = jnp.maximum(_dot(h_ref[...], w1_ref[...]) + b1_ref[...], 0.0)
    o_ref[...] = x + _dot(y, w2_ref[...]) + b2_ref[...]


def _ffn(x, h, w1, b1, w2, b2):
    x2 = x.reshape(B * SQ, D)
    h2 = h.reshape(B * SQ, D)
    out = pl.pallas_call(
        _ffn_kernel,
        grid=(B * SQ // RB,),
        in_specs=[
            pl.BlockSpec((RB, D), lambda i: (i, 0)),
            pl.BlockSpec((RB, D), lambda i: (i, 0)),
            pl.BlockSpec((D, FF), lambda i: (0, 0)),
            pl.BlockSpec((1, FF), lambda i: (0, 0)),
            pl.BlockSpec((FF, D), lambda i: (0, 0)),
            pl.BlockSpec((1, D), lambda i: (0, 0)),
        ],
        out_specs=pl.BlockSpec((RB, D), lambda i: (i, 0)),
        out_shape=jax.ShapeDtypeStruct((B * SQ, D), F32),
    )(x2, h2, w1, b1, w2, b2)
    return out.reshape(B, SQ, D)


def _head_kernel(x_ref, wm_ref, bm_ref, ws_ref, bs_ref, mel_ref, stop_ref):
    h = _ln(x_ref[...])
    mel_ref[...] = _dot(h, wm_ref[...]) + bm_ref[...]
    stop_ref[...] = _dot(h, ws_ref[...]) + bs_ref[...]


def _heads(x, wm, bm, ws, bs):
    x2 = x.reshape(B * SQ, D)
    mel, stop = pl.pallas_call(
        _head_kernel,
        grid=(B * SQ // RB,),
        in_specs=[
            pl.BlockSpec((RB, D), lambda i: (i, 0)),
            pl.BlockSpec((D, NMEL), lambda i: (0, 0)),
            pl.BlockSpec((1, NMEL), lambda i: (0, 0)),
            pl.BlockSpec((D, 1), lambda i: (0, 0)),
            pl.BlockSpec((1, 1), lambda i: (0, 0)),
        ],
        out_specs=[
            pl.BlockSpec((RB, NMEL), lambda i: (i, 0)),
            pl.BlockSpec((RB, 1), lambda i: (i, 0)),
        ],
        out_shape=[
            jax.ShapeDtypeStruct((B * SQ, NMEL), F32),
            jax.ShapeDtypeStruct((B * SQ, 1), F32),
        ],
    )(x2, wm, bm, ws, bs)
    return mel.reshape(B, SQ, NMEL), stop.reshape(B, SQ, 1)


def _ln_ref(x):
    m = jnp.mean(x, axis=-1, keepdims=True)
    v = jnp.var(x, axis=-1, keepdims=True)
    return (x - m) / jnp.sqrt(v + 1e-5)


def _sin_pe():
    pos = jnp.arange(SQ, dtype=F32)[:, None]
    div = jnp.exp(jnp.arange(0, D, 2, dtype=F32) * (-np.log(10000.0) / D))
    pe = jnp.zeros((SQ, D), dtype=F32)
    pe = pe.at[:, 0::2].set(jnp.sin(pos * div))
    pe = pe.at[:, 1::2].set(jnp.cos(pos * div))
    return pe


def kernel(input_, keys, pre_W1, pre_b1, pre_W2, pre_b2, alpha, Wqk, Wv,
           Wo_s, rot, Wq_c, Wk_c, Wv_c, Wo_c, Wf1, bf1, Wf2, bf2, Wm, bm,
           Ws, bs):
    pe_scaled = alpha * _sin_pe()
    x = _prenet(input_, pre_W1, pre_b1.reshape(1, PRE_H), pre_W2,
                pre_b2.reshape(1, D), pe_scaled)
    attns = []
    pos = jnp.arange(SQ)
    for l in range(L):
        qk4, v4 = _proj(x, Wqk[l], Wv[l])
        h = _ln_ref(x)
        qkr = (h @ Wqk[l]).reshape(B, SQ, H, DH).transpose(0, 2, 1, 3)
        rotated = jnp.einsum('bhsd,hdn->bhsn', qkr, rot[l])
        buckets = jnp.argmax(
            jnp.concatenate([rotated, -rotated], axis=-1), axis=-1)
        ticker = jnp.argsort(buckets * SQ + pos[None, None, :], axis=-1)
        ranks = jnp.argsort(ticker, axis=-1)
        rk = jnp.broadcast_to(ranks[..., None].astype(F32), (B, H, SQ, 8))
        x = _lsh_attn(qk4, v4, rk, x, Wo_s[l])
        x, a = _cross(x, _ln_ref(x), keys, Wq_c[l], Wk_c[l], Wv_c[l], Wo_c[l])
        attns.append(a)
        x = _ffn(x, _ln_ref(x), Wf1[l], bf1[l].reshape(1, FF),
                 Wf2[l], bf2[l].reshape(1, D))
    mel, stop = _heads(x, Wm, bm.reshape(1, NMEL), Ws, bs.reshape(1, 1))
    return (mel, stop, jnp.stack(attns, axis=0))
